# baseline (speedup 1.0000x reference)
import functools

import numpy as np

import jax
import jax.numpy as jnp
from jax import lax
from jax.experimental import pallas as pl
from jax.experimental.pallas import tpu as pltpu
from jax.experimental.pallas import tpu_sc as plsc

_N = 10000
_E = 320000
_V = 3
_DIN = 128
_H1 = 64
_H2 = 32

_NC = 2
_NS = 16
_NW = _NC * _NS

_CHUNK = 200
_EPW = _E // _NW
_NCHUNKS = _EPW // _CHUNK
_ZROWS = _N // _NS


_BN = 2000


def _mm_body(x_ref, w_ref, o_ref):
    o_ref[0] = jnp.dot(
        x_ref[...], w_ref[0], preferred_element_type=jnp.float32
    ).astype(jnp.bfloat16)


def _support_matmul(x, W_views):
    return pl.pallas_call(
        _mm_body,
        grid=(_V, _N // _BN),
        in_specs=[
            pl.BlockSpec((_BN, _DIN), lambda v, i: (i, 0)),
            pl.BlockSpec((1, _DIN, _H1), lambda v, i: (v, 0, 0)),
        ],
        out_specs=pl.BlockSpec((1, _BN, _H1), lambda v, i: (v, i, 0)),
        out_shape=jax.ShapeDtypeStruct((_V, _N, _H1), jnp.bfloat16),
    )(x, W_views)


_BF = 1000


def _fuse_body(p_ref, w_ref, o_ref):
    p = p_ref[...]
    h = jax.nn.relu(p[0, :, :_H1] + p[0, :, _H1:])
    h += jax.nn.relu(p[1, :, :_H1] + p[1, :, _H1:])
    h += jax.nn.relu(p[2, :, :_H1] + p[2, :, _H1:])
    mean = h * (1.0 / _V)
    o_ref[...] = jnp.dot(
        mean, w_ref[...], preferred_element_type=jnp.float32
    ).astype(jnp.bfloat16)


def _fuse_matmul(p1, W_out):
    return pl.pallas_call(
        _fuse_body,
        grid=(_N // _BF,),
        in_specs=[
            pl.BlockSpec((_V, _BF, _NC * _H1), lambda i: (0, i, 0)),
            pl.BlockSpec((_H1, _H2), lambda i: (0, 0)),
        ],
        out_specs=pl.BlockSpec((_BF, _H2), lambda i: (i, 0)),
        out_shape=jax.ShapeDtypeStruct((_N, _H2), jnp.bfloat16),
    )(p1, W_out)


def _final_body(p_ref, o_ref):
    p = p_ref[...]
    o_ref[...] = jax.nn.relu(p[:, :_H2] + p[:, _H2:])


def _final_relu(p2):
    return pl.pallas_call(
        _final_body,
        grid=(_N // _BN,),
        in_specs=[pl.BlockSpec((_BN, _NC * _H2), lambda i: (i, 0))],
        out_specs=pl.BlockSpec((_BN, _H2), lambda i: (i, 0)),
        out_shape=jax.ShapeDtypeStruct((_N, _H2), jnp.float32),
    )(p2)


def _make_sc_scatter(view_ids, H):
    n_views = len(view_ids)
    mesh = plsc.VectorSubcoreMesh(core_axis_name="c", subcore_axis_name="s")

    zoffs = list(range(0, _ZROWS - _CHUNK + 1, _CHUNK))
    if zoffs[-1] != _ZROWS - _CHUNK:
        zoffs.append(_ZROWS - _CHUNK)

    scratch = [
        pltpu.VMEM((_EPW,), jnp.int32),
        pltpu.VMEM((_EPW,), jnp.int32),
        pltpu.VMEM((_EPW,), jnp.float32),
        pltpu.VMEM((_CHUNK, H), jnp.bfloat16),
        pltpu.VMEM((_CHUNK, H), jnp.bfloat16),
        pltpu.VMEM((_CHUNK, H), jnp.float32),
        pltpu.VMEM((_CHUNK, H), jnp.float32),
        pltpu.VMEM((_CHUNK, H), jnp.float32),
        pltpu.VMEM_SHARED((_N, H), jnp.float32),
        pltpu.SemaphoreType.DMA,
        pltpu.SemaphoreType.DMA,
        pltpu.SemaphoreType.DMA,
        pltpu.SemaphoreType.DMA,
    ]

    def body(ei_ref, ew_ref, sup3_ref, out_ref, *rest):
        (src_v, dst_v, w_v, g0, g1, s0, s1, zbuf, acc,
         smg0, smg1, sms0, sms1) = rest
        gbufs = ((g0, smg0), (g1, smg1))
        sbufs = ((s0, sms0), (s1, sms1))

        c = lax.axis_index("c")
        s = lax.axis_index("s")
        wid = c * _NS + s
        zbase = s * _ZROWS

        zero16 = jnp.zeros((16,), jnp.float32)

        @plsc.parallel_loop(0, _CHUNK, step=1, unroll=4)
        def zfill(r):
            for f in range(H // 16):
                zbuf[r, pl.ds(f * 16, 16)] = zero16

        def fire_gather(sup, p, k):
            gb, sem = gbufs[p]
            pltpu.async_copy(sup.at[src_v.at[pl.ds(k * _CHUNK, _CHUNK)]], gb, sem)

        def wait_gather(sup, p, k):
            gb, sem = gbufs[p]
            pltpu.make_async_copy(
                sup.at[src_v.at[pl.ds(k * _CHUNK, _CHUNK)]], gb, sem
            ).wait()

        def fire_scatter(p, k):
            sb, sem = sbufs[p]
            pltpu.async_copy(
                sb, acc.at[dst_v.at[pl.ds(k * _CHUNK, _CHUNK)]], sem, add=True
            )

        def wait_scatter(p, k):
            sb, sem = sbufs[p]
            pltpu.make_async_copy(
                sb, acc.at[dst_v.at[pl.ds(k * _CHUNK, _CHUNK)]], sem
            ).wait()

        def scale_chunk(p, k):
            gb, _ = gbufs[p]
            sb, _ = sbufs[p]
            kbase = k * _CHUNK

            @plsc.parallel_loop(0, _CHUNK, step=1, unroll=4)
            def scale(e):
                wt = plsc.load_gather(w_v, [jnp.full((16,), kbase + e, jnp.int32)])
                for f in range(H // 32):
                    ab = gb[e, pl.ds(32 * f, 32)]
                    a, b = plsc.unpack(ab, format=plsc.PackFormat.INTERLEAVED)
                    sb[e, pl.ds(32 * f, 16)] = a * wt
                    sb[e, pl.ds(32 * f + 16, 16)] = b * wt

        for i, vi in enumerate(view_ids):
            sup = sup3_ref.at[vi] if n_views > 1 else sup3_ref
            for off in zoffs:
                pltpu.sync_copy(zbuf, acc.at[pl.ds(zbase + off, _CHUNK)])

            pltpu.sync_copy(
                ei_ref.at[pl.ds((2 * vi + 0) * _E + wid * _EPW, _EPW)], src_v)
            pltpu.sync_copy(
                ei_ref.at[pl.ds((2 * vi + 1) * _E + wid * _EPW, _EPW)], dst_v)
            pltpu.sync_copy(ew_ref.at[pl.ds(vi * _E + wid * _EPW, _EPW)], w_v)

            plsc.subcore_barrier()

            fire_gather(sup, 0, 0)
            fire_gather(sup, 1, 1)
            for k in (0, 1):
                wait_gather(sup, k, k)
                scale_chunk(k, k)
                fire_gather(sup, k, k + 2)
                fire_scatter(k, k)

            def pair(gi, _):
                k0 = 2 * gi
                for p in range(2):
                    k = k0 + p
                    wait_gather(sup, p, k)
                    wait_scatter(p, k - 2)
                    scale_chunk(p, k)

                    @pl.when(k + 2 < _NCHUNKS)
                    def _():
                        fire_gather(sup, p, k + 2)

                    fire_scatter(p, k)
                return 0

            lax.fori_loop(1, _NCHUNKS // 2, pair, 0)

            wait_scatter(0, _NCHUNKS - 2)
            wait_scatter(1, _NCHUNKS - 1)

            plsc.subcore_barrier()

            dst = (out_ref.at[i] if n_views > 1 else out_ref)
            pltpu.sync_copy(
                acc.at[pl.ds(zbase, _ZROWS)],
                dst.at[pl.ds(zbase, _ZROWS), pl.ds(c * H, H)],
            )

    return pl.kernel(
        body,
        out_type=jax.ShapeDtypeStruct(
            (n_views, _N, _NC * H) if n_views > 1 else (_N, _NC * H),
            jnp.float32),
        mesh=mesh,
        scratch_types=scratch,
        compiler_params=pltpu.CompilerParams(
            use_tc_tiling_on_sc=False, needs_layout_passes=False
        ),
    )


def _slot_perm(H):
    perm = []
    for f in range(H // 32):
        perm += list(range(32 * f, 32 * f + 32, 2))
        perm += list(range(32 * f + 1, 32 * f + 32, 2))
    return perm


def kernel(x, edge_index, edge_weight, W_views, W_out):
    ei_flat = edge_index.reshape(-1)
    ew_flat = edge_weight.reshape(-1)

    support = _support_matmul(x, W_views)

    sc1 = _make_sc_scatter([0, 1, 2], _H1)
    p1 = sc1(ei_flat, ew_flat, support)

    w_pp = W_out[np.array(_slot_perm(_H1))][:, np.argsort(_slot_perm(_H2))]
    s2 = _fuse_matmul(p1, w_pp)

    sc2 = _make_sc_scatter([0], _H2)
    p2 = sc2(ei_flat, ew_flat, s2)

    return _final_relu(p2)


# speedup vs baseline: 1.0004x; 1.0004x over previous
import numpy as np

import jax
import jax.numpy as jnp
from jax import lax
from jax.experimental import pallas as pl
from jax.experimental.pallas import tpu as pltpu
from jax.experimental.pallas import tpu_sc as plsc

_N = 10000
_E = 320000
_V = 3
_DIN = 128
_H1 = 64
_H2 = 32

_NC = 2
_NS = 16
_NW = _NC * _NS

_CHUNK = 200
_EPW = _E // _NW
_NCHUNKS = _EPW // _CHUNK
_ZROWS = _N // _NS


_BN = 2000


def _mm_body(x_ref, w_ref, o_ref):
    o_ref[0] = jnp.dot(
        x_ref[...], w_ref[0], preferred_element_type=jnp.float32
    ).astype(jnp.bfloat16)


def _support_matmul(x, W_views):
    return pl.pallas_call(
        _mm_body,
        grid=(_V, _N // _BN),
        in_specs=[
            pl.BlockSpec((_BN, _DIN), lambda v, i: (i, 0)),
            pl.BlockSpec((1, _DIN, _H1), lambda v, i: (v, 0, 0)),
        ],
        out_specs=pl.BlockSpec((1, _BN, _H1), lambda v, i: (v, i, 0)),
        out_shape=jax.ShapeDtypeStruct((_V, _N, _H1), jnp.bfloat16),
    )(x, W_views)


_BF = 1000


def _fuse_body(p_ref, w_ref, o_ref):
    p = p_ref[...]
    h = jax.nn.relu(p[0, :, :_H1] + p[0, :, _H1:])
    h += jax.nn.relu(p[1, :, :_H1] + p[1, :, _H1:])
    h += jax.nn.relu(p[2, :, :_H1] + p[2, :, _H1:])
    mean = h * (1.0 / _V)
    o_ref[...] = jnp.dot(
        mean, w_ref[...], preferred_element_type=jnp.float32
    ).astype(jnp.bfloat16)


def _fuse_matmul(p1, W_out):
    return pl.pallas_call(
        _fuse_body,
        grid=(_N // _BF,),
        in_specs=[
            pl.BlockSpec((_V, _BF, _NC * _H1), lambda i: (0, i, 0)),
            pl.BlockSpec((_H1, _H2), lambda i: (0, 0)),
        ],
        out_specs=pl.BlockSpec((_BF, _H2), lambda i: (i, 0)),
        out_shape=jax.ShapeDtypeStruct((_N, _H2), jnp.bfloat16),
    )(p1, W_out)


def _final_body(p_ref, o_ref):
    p = p_ref[...]
    o_ref[...] = jax.nn.relu(p[:, :_H2] + p[:, _H2:])


def _final_relu(p2):
    return pl.pallas_call(
        _final_body,
        grid=(_N // _BN,),
        in_specs=[pl.BlockSpec((_BN, _NC * _H2), lambda i: (i, 0))],
        out_specs=pl.BlockSpec((_BN, _H2), lambda i: (i, 0)),
        out_shape=jax.ShapeDtypeStruct((_N, _H2), jnp.float32),
    )(p2)


def _make_sc_scatter(view_ids, H):
    n_views = len(view_ids)
    mesh = plsc.VectorSubcoreMesh(core_axis_name="c", subcore_axis_name="s")

    zoffs = list(range(0, _ZROWS - _CHUNK + 1, _CHUNK))
    if zoffs[-1] != _ZROWS - _CHUNK:
        zoffs.append(_ZROWS - _CHUNK)

    scratch = [
        pltpu.VMEM((_EPW,), jnp.int32),
        pltpu.VMEM((_EPW,), jnp.int32),
        pltpu.VMEM((_EPW,), jnp.float32),
        pltpu.VMEM((_CHUNK, H), jnp.bfloat16),
        pltpu.VMEM((_CHUNK, H), jnp.bfloat16),
        pltpu.VMEM((_CHUNK, H), jnp.float32),
        pltpu.VMEM((_CHUNK, H), jnp.float32),
        pltpu.VMEM((_CHUNK, H), jnp.float32),
        pltpu.VMEM_SHARED((_N, H), jnp.float32),
        pltpu.SemaphoreType.DMA,
        pltpu.SemaphoreType.DMA,
        pltpu.SemaphoreType.DMA,
        pltpu.SemaphoreType.DMA,
    ]

    def body(ei_ref, ew_ref, sup3_ref, out_ref, *rest):
        (src_v, dst_v, w_v, g0, g1, s0, s1, zbuf, acc,
         smg0, smg1, sms0, sms1) = rest
        gbufs = ((g0, smg0), (g1, smg1))
        sbufs = ((s0, sms0), (s1, sms1))

        c = lax.axis_index("c")
        s = lax.axis_index("s")
        wid = c * _NS + s
        zbase = s * _ZROWS

        zero16 = jnp.zeros((16,), jnp.float32)

        @plsc.parallel_loop(0, _CHUNK, step=1, unroll=4)
        def zfill(r):
            for f in range(H // 16):
                zbuf[r, pl.ds(f * 16, 16)] = zero16

        def fire_gather(sup, p, k):
            gb, sem = gbufs[p]
            pltpu.async_copy(sup.at[src_v.at[pl.ds(k * _CHUNK, _CHUNK)]], gb, sem)

        def wait_gather(sup, p, k):
            gb, sem = gbufs[p]
            pltpu.make_async_copy(
                sup.at[src_v.at[pl.ds(k * _CHUNK, _CHUNK)]], gb, sem
            ).wait()

        def fire_scatter(p, k):
            sb, sem = sbufs[p]
            pltpu.async_copy(
                sb, acc.at[dst_v.at[pl.ds(k * _CHUNK, _CHUNK)]], sem, add=True
            )

        def wait_scatter(p, k):
            sb, sem = sbufs[p]
            pltpu.make_async_copy(
                sb, acc.at[dst_v.at[pl.ds(k * _CHUNK, _CHUNK)]], sem
            ).wait()

        def scale_chunk(p, k):
            gb, _ = gbufs[p]
            sb, _ = sbufs[p]
            kbase = k * _CHUNK

            @plsc.parallel_loop(0, _CHUNK, step=1, unroll=4)
            def scale(e):
                wt = plsc.load_gather(w_v, [jnp.full((16,), kbase + e, jnp.int32)])
                for f in range(H // 32):
                    ab = gb[e, pl.ds(32 * f, 32)]
                    a, b = plsc.unpack(ab, format=plsc.PackFormat.INTERLEAVED)
                    sb[e, pl.ds(32 * f, 16)] = a * wt
                    sb[e, pl.ds(32 * f + 16, 16)] = b * wt

        for i, vi in enumerate(view_ids):
            sup = sup3_ref.at[vi] if n_views > 1 else sup3_ref
            for off in zoffs:
                pltpu.sync_copy(zbuf, acc.at[pl.ds(zbase + off, _CHUNK)])

            pltpu.sync_copy(
                ei_ref.at[pl.ds((2 * vi + 0) * _E + wid * _EPW, _EPW)], src_v)
            pltpu.sync_copy(
                ei_ref.at[pl.ds((2 * vi + 1) * _E + wid * _EPW, _EPW)], dst_v)
            pltpu.sync_copy(ew_ref.at[pl.ds(vi * _E + wid * _EPW, _EPW)], w_v)

            plsc.subcore_barrier()

            fire_gather(sup, 0, 0)
            fire_gather(sup, 1, 1)
            for k in (0, 1):
                wait_gather(sup, k, k)
                scale_chunk(k, k)
                fire_gather(sup, k, k + 2)
                fire_scatter(k, k)

            def pair(gi, _):
                k0 = 2 * gi
                for p in range(2):
                    k = k0 + p
                    wait_gather(sup, p, k)
                    wait_scatter(p, k - 2)
                    scale_chunk(p, k)

                    @pl.when(k + 2 < _NCHUNKS)
                    def _():
                        fire_gather(sup, p, k + 2)

                    fire_scatter(p, k)
                return 0

            lax.fori_loop(1, _NCHUNKS // 2, pair, 0)

            wait_scatter(0, _NCHUNKS - 2)
            wait_scatter(1, _NCHUNKS - 1)

            plsc.subcore_barrier()

            dst = (out_ref.at[i] if n_views > 1 else out_ref)
            pltpu.sync_copy(
                acc.at[pl.ds(zbase, _ZROWS)],
                dst.at[pl.ds(zbase, _ZROWS), pl.ds(c * H, H)],
            )

    return pl.kernel(
        body,
        out_type=jax.ShapeDtypeStruct(
            (n_views, _N, _NC * H) if n_views > 1 else (_N, _NC * H),
            jnp.float32),
        mesh=mesh,
        scratch_types=scratch,
        compiler_params=pltpu.CompilerParams(
            use_tc_tiling_on_sc=False, needs_layout_passes=False
        ),
    )


def _slot_perm(H):
    perm = []
    for f in range(H // 32):
        perm += list(range(32 * f, 32 * f + 32, 2))
        perm += list(range(32 * f + 1, 32 * f + 32, 2))
    return perm


def kernel(x, edge_index, edge_weight, W_views, W_out):
    ei_flat = edge_index.reshape(-1)
    ew_flat = edge_weight.reshape(-1)

    support = _support_matmul(x, W_views)

    sc1 = _make_sc_scatter([0, 1, 2], _H1)
    p1 = sc1(ei_flat, ew_flat, support)

    w_pp = W_out[np.array(_slot_perm(_H1))][:, np.argsort(_slot_perm(_H2))]
    s2 = _fuse_matmul(p1, w_pp)

    sc2 = _make_sc_scatter([0], _H2)
    p2 = sc2(ei_flat, ew_flat, s2)

    return _final_relu(p2)
